# baseline (speedup 1.0000x reference)
import functools

import jax
import jax.numpy as jnp
from jax import lax
from jax.experimental import pallas as pl
from jax.experimental.pallas import tpu as pltpu
from jax.experimental.pallas import tpu_sc as plsc

_VOCAB = 819
_D = 128
_VPAD = 824

_NC = 2
_NS = 16
_NW = _NC * _NS

_NBATCH = 16384
_ROWS = 26
_RPAD = 32
_BAT_PW = _NBATCH // _NW


def _mlp_table_body(emb_ref, w1t_ref, b1_ref, w2t_ref, b2_ref, out_ref):
    inv_sqrt2 = 0.7071067811865476
    x = emb_ref[...]
    x = x * 0.5 * (1.0 + lax.erf(x * inv_sqrt2))
    x = jnp.dot(x, w1t_ref[...], preferred_element_type=jnp.float32) + b1_ref[...]
    x = x * 0.5 * (1.0 + lax.erf(x * inv_sqrt2))
    x = jnp.dot(x, w2t_ref[...], preferred_element_type=jnp.float32) + b2_ref[...]
    out_ref[...] = x


def _transform_table(emb, W1, b1, W2, b2):
    emb_pad = jnp.pad(emb, ((0, _VPAD - _VOCAB), (0, 0)))
    return pl.pallas_call(
        _mlp_table_body,
        out_shape=jax.ShapeDtypeStruct((_VPAD, _D), jnp.float32),
    )(emb_pad, W1.T, b1.reshape(1, _D), W2.T, b2.reshape(1, _D))


_NBUF = 8


_NSPLIT = 2
_BAT_SPLIT = _NBATCH // _NSPLIT
_BAT_PW_S = _BAT_SPLIT // _NW


def _gather_body(table_hbm, idx_hbm, out_hbm, idx_v, rows_v, gsems, ssems):
    c = lax.axis_index("c")
    s = lax.axis_index("s")
    wid = s * _NC + c
    base = wid * _BAT_PW_S
    pltpu.sync_copy(
        idx_hbm.at[pl.ds(wid * _BAT_PW_S * _RPAD, _BAT_PW_S * _RPAD)], idx_v
    )

    def gather_args(j, b):
        return (
            table_hbm.at[idx_v.at[pl.ds(j * _RPAD, _ROWS)]],
            rows_v.at[b],
            gsems[b],
        )

    def scatter_args(j, b):
        return rows_v.at[b], out_hbm.at[base + j], ssems[b]

    for b in range(_NBUF):
        pltpu.async_copy(*gather_args(b, b))

    def outer(i, carry):
        jo = i * _NBUF
        for b in range(_NBUF):
            pltpu.make_async_copy(*gather_args(jo + b, b)).wait()
            pltpu.async_copy(*scatter_args(jo + b, b))
        for b in range(_NBUF):
            jn = jo + b + _NBUF

            @pl.when(jn < _BAT_PW_S)
            def _():
                pltpu.make_async_copy(*scatter_args(jo + b, b)).wait()
                pltpu.async_copy(*gather_args(jn, b))

        return carry

    lax.fori_loop(0, _BAT_PW_S // _NBUF, outer, 0)
    for b in range(_NBUF):
        pltpu.make_async_copy(*scatter_args(_BAT_PW_S - _NBUF + b, b)).wait()


@functools.lru_cache(maxsize=1)
def _gather_call():
    return pl.kernel(
        _gather_body,
        out_type=jax.ShapeDtypeStruct((_BAT_SPLIT, _ROWS, _D), jnp.float32),
        mesh=plsc.VectorSubcoreMesh(core_axis_name="c", subcore_axis_name="s"),
        scratch_types=[
            pltpu.VMEM((_BAT_PW_S * _RPAD,), jnp.int32),
            pltpu.VMEM((_NBUF, _ROWS, _D), jnp.float32),
            [pltpu.SemaphoreType.DMA] * _NBUF,
            [pltpu.SemaphoreType.DMA] * _NBUF,
        ],
        compiler_params=pltpu.CompilerParams(
            use_tc_tiling_on_sc=True, needs_layout_passes=True
        ),
    )


def kernel(X, emb, W1, b1, W2, b2):
    table = _transform_table(emb, W1, b1, W2, b2)
    idx = jnp.pad(X.astype(jnp.int32), ((0, 0), (0, _RPAD - _ROWS))).reshape(
        _NSPLIT, _BAT_SPLIT * _RPAD
    )
    call = _gather_call()
    parts = [call(table, idx[p]) for p in range(_NSPLIT)]
    return jnp.concatenate(parts, axis=0)


# speedup vs baseline: 2.1341x; 2.1341x over previous
import functools

import jax
import jax.numpy as jnp
from jax import lax
from jax.experimental import pallas as pl
from jax.experimental.pallas import tpu as pltpu
from jax.experimental.pallas import tpu_sc as plsc

_VOCAB = 819
_D = 128
_VPAD = 824

_NC = 2
_NS = 16
_NW = _NC * _NS

_NBATCH = 16384
_ROWS = 26
_RPAD = 32
_BAT_PW = _NBATCH // _NW


def _mlp_table_body(emb_ref, w1t_ref, b1_ref, w2t_ref, b2_ref, out_ref):
    inv_sqrt2 = 0.7071067811865476
    x = emb_ref[...]
    x = x * 0.5 * (1.0 + lax.erf(x * inv_sqrt2))
    x = jnp.dot(x, w1t_ref[...], preferred_element_type=jnp.float32) + b1_ref[...]
    x = x * 0.5 * (1.0 + lax.erf(x * inv_sqrt2))
    x = jnp.dot(x, w2t_ref[...], preferred_element_type=jnp.float32) + b2_ref[...]
    out_ref[...] = x


def _transform_table(emb, W1, b1, W2, b2):
    emb_pad = jnp.pad(emb, ((0, _VPAD - _VOCAB), (0, 0)))
    return pl.pallas_call(
        _mlp_table_body,
        out_shape=jax.ShapeDtypeStruct((_VPAD, _D), jnp.float32),
    )(emb_pad, W1.T, b1.reshape(1, _D), W2.T, b2.reshape(1, _D))


_NBUF = 8


def _gather_body(table_hbm, idx_hbm, out_hbm, table_sp, idx_v, rows_v, gsems, ssems):
    c = lax.axis_index("c")
    s = lax.axis_index("s")
    wid = s * _NC + c
    base = wid * _BAT_PW
    @pl.when(s == 0)
    def _():
        pltpu.sync_copy(table_hbm, table_sp)

    pltpu.sync_copy(
        idx_hbm.at[pl.ds(wid * _BAT_PW * _RPAD, _BAT_PW * _RPAD)], idx_v
    )
    plsc.subcore_barrier()

    def gather_args(j, b):
        return (
            table_sp.at[idx_v.at[pl.ds(j * _RPAD, _ROWS)]],
            rows_v.at[b],
            gsems[b],
        )

    def scatter_args(j, b):
        return rows_v.at[b], out_hbm.at[base + j], ssems[b]

    for b in range(_NBUF):
        pltpu.async_copy(*gather_args(b, b))

    def outer(i, carry):
        jo = i * _NBUF
        for b in range(_NBUF):
            pltpu.make_async_copy(*gather_args(jo + b, b)).wait()
            pltpu.async_copy(*scatter_args(jo + b, b))
        for b in range(_NBUF):
            jn = jo + b + _NBUF

            @pl.when(jn < _BAT_PW)
            def _():
                pltpu.make_async_copy(*scatter_args(jo + b, b)).wait()
                pltpu.async_copy(*gather_args(jn, b))

        return carry

    lax.fori_loop(0, _BAT_PW // _NBUF, outer, 0)
    for b in range(_NBUF):
        pltpu.make_async_copy(*scatter_args(_BAT_PW - _NBUF + b, b)).wait()


@functools.lru_cache(maxsize=1)
def _gather_call():
    return pl.kernel(
        _gather_body,
        out_type=jax.ShapeDtypeStruct((_NBATCH, _ROWS, _D), jnp.float32),
        mesh=plsc.VectorSubcoreMesh(core_axis_name="c", subcore_axis_name="s"),
        scratch_types=[
            pltpu.VMEM_SHARED((_VPAD, _D), jnp.float32),
            pltpu.VMEM((_BAT_PW * _RPAD,), jnp.int32),
            pltpu.VMEM((_NBUF, _ROWS, _D), jnp.float32),
            [pltpu.SemaphoreType.DMA] * _NBUF,
            [pltpu.SemaphoreType.DMA] * _NBUF,
        ],
        compiler_params=pltpu.CompilerParams(
            use_tc_tiling_on_sc=True, needs_layout_passes=True
        ),
    )


def kernel(X, emb, W1, b1, W2, b2):
    table = _transform_table(emb, W1, b1, W2, b2)
    idx = jnp.pad(X.astype(jnp.int32), ((0, 0), (0, _RPAD - _ROWS))).reshape(-1)
    return _gather_call()(table, idx)


# speedup vs baseline: 5.7161x; 2.6785x over previous
import functools

import jax
import jax.numpy as jnp
from jax import lax
from jax.experimental import pallas as pl
from jax.experimental.pallas import tpu as pltpu
from jax.experimental.pallas import tpu_sc as plsc

_VOCAB = 819
_D = 128
_VPAD = 824

_NC = 2
_NS = 16
_NW = _NC * _NS

_NBATCH = 16384
_ROWS = 26
_BAT_PW = _NBATCH // _NW


def _mlp_table_body(emb_ref, w1t_ref, b1_ref, w2t_ref, b2_ref, out_ref):
    inv_sqrt2 = 0.7071067811865476
    x = emb_ref[...]
    x = x * 0.5 * (1.0 + lax.erf(x * inv_sqrt2))
    x = jnp.dot(x, w1t_ref[...], preferred_element_type=jnp.float32) + b1_ref[...]
    x = x * 0.5 * (1.0 + lax.erf(x * inv_sqrt2))
    x = jnp.dot(x, w2t_ref[...], preferred_element_type=jnp.float32) + b2_ref[...]
    out_ref[...] = x


def _transform_table(emb, W1, b1, W2, b2):
    emb_pad = jnp.pad(emb, ((0, _VPAD - _VOCAB), (0, 0)))
    return pl.pallas_call(
        _mlp_table_body,
        out_shape=jax.ShapeDtypeStruct((_VPAD, _D), jnp.float32),
    )(emb_pad, W1.T, b1.reshape(1, _D), W2.T, b2.reshape(1, _D))


_NBUF = 8
_CHUNK = 64
_IPW = _BAT_PW * _ROWS
_NCHUNK = _IPW // _CHUNK
_B = _NBATCH * _ROWS


def _gather_body(table_hbm, idx_hbm, out_hbm, table_sp, idx_v, rows_v, gsems, ssems):
    c = lax.axis_index("c")
    s = lax.axis_index("s")
    wid = s * _NC + c
    @pl.when(s == 0)
    def _():
        pltpu.sync_copy(table_hbm, table_sp)

    pltpu.sync_copy(idx_hbm.at[pl.ds(wid * _IPW, _IPW)], idx_v)
    plsc.subcore_barrier()

    def gather_args(ch, b):
        return (
            table_sp.at[idx_v.at[pl.ds(ch * _CHUNK, _CHUNK)]],
            rows_v.at[b],
            gsems[b],
        )

    def scatter_args(i, b):
        off = i * _NBATCH + wid * _BAT_PW + b * _CHUNK
        return rows_v.at[b], out_hbm.at[pl.ds(off, _CHUNK)], ssems[b]

    for b in range(_NBUF):
        pltpu.async_copy(*gather_args(b, b))

    def outer(i, carry):
        co = i * _NBUF
        for b in range(_NBUF):
            pltpu.make_async_copy(*gather_args(co + b, b)).wait()
            pltpu.async_copy(*scatter_args(i, b))
        for b in range(_NBUF):
            cn = co + b + _NBUF

            @pl.when(cn < _NCHUNK)
            def _():
                pltpu.make_async_copy(*scatter_args(i, b)).wait()
                pltpu.async_copy(*gather_args(cn, b))

        return carry

    lax.fori_loop(0, _NCHUNK // _NBUF, outer, 0)
    for b in range(_NBUF):
        pltpu.make_async_copy(*scatter_args(_NCHUNK // _NBUF - 1, b)).wait()


@functools.lru_cache(maxsize=1)
def _gather_call():
    return pl.kernel(
        _gather_body,
        out_type=jax.ShapeDtypeStruct((_B, _D), jnp.float32),
        mesh=plsc.VectorSubcoreMesh(core_axis_name="c", subcore_axis_name="s"),
        scratch_types=[
            pltpu.VMEM_SHARED((_VPAD, _D), jnp.float32),
            pltpu.VMEM((_IPW,), jnp.int32),
            pltpu.VMEM((_NBUF, _CHUNK, _D), jnp.float32),
            [pltpu.SemaphoreType.DMA] * _NBUF,
            [pltpu.SemaphoreType.DMA] * _NBUF,
        ],
        compiler_params=pltpu.CompilerParams(
            use_tc_tiling_on_sc=True, needs_layout_passes=True
        ),
    )


def kernel(X, emb, W1, b1, W2, b2):
    table = _transform_table(emb, W1, b1, W2, b2)
    idx = (
        X.astype(jnp.int32)
        .reshape(_NW, _BAT_PW, _ROWS)
        .transpose(0, 2, 1)
        .reshape(-1)
    )
    out = _gather_call()(table, idx)
    return out.reshape(_ROWS, _NBATCH, _D).transpose(1, 0, 2)
